# Initial kernel scaffold; baseline (speedup 1.0000x reference)
import jax
import jax.numpy as jnp
from jax.experimental import pallas as pl


def kernel(x, edge_index, W_src0, W_dst0, att_src0, att_dst0, b0, W_src1, W_dst1, att_src1, att_dst1, b1):
    raise NotImplementedError("write your pallas kernel here")


# speedup vs baseline: 23.9561x; 23.9561x over previous
import jax
import jax.numpy as jnp
from jax import lax
from jax.experimental import pallas as pl
from jax.experimental.pallas import tpu as pltpu
from jax.experimental.pallas import tpu_sc as plsc

N_NODES = 10000
N_PAD = 10240
N_EDGES = 320000
D = 128
DE = 16
NC = 2
NS = 16
NHALF = N_PAD // NC
NQ = NHALF // 2
EPT = N_EDGES // NS
SBLK = 2000
NBLK = EPT // SBLK
K = 80
EBUF = EPT + 3 * K
ROWS_ACC = NQ + DE
SUB_PT = NQ // NS


def _prep_body(x_ref, ws_ref, wd_ref, ats_ref, atd_ref, h_ref, a2_ref):
    xb = x_ref[...]
    hs = jnp.dot(xb, ws_ref[...], preferred_element_type=jnp.float32)
    hd = jnp.dot(xb, wd_ref[...], preferred_element_type=jnp.float32)
    h_ref[...] = hs
    a_s = jnp.sum(hs * ats_ref[...][None, :], axis=1)
    a_d = jnp.sum(hd * atd_ref[...][None, :], axis=1)
    a2_ref[...] = jnp.concatenate(
        [a_s[None, :], a_d[None, :], jnp.zeros((6, a_s.shape[0]), jnp.float32)], axis=0)


def _tc_prep(x_p, W_src, W_dst, att_src, att_dst):
    grid = N_PAD // 512
    return pl.pallas_call(
        _prep_body,
        grid=(grid,),
        in_specs=[
            pl.BlockSpec((512, D), lambda j: (j, 0)),
            pl.BlockSpec((D, D), lambda j: (0, 0)),
            pl.BlockSpec((D, D), lambda j: (0, 0)),
            pl.BlockSpec((D,), lambda j: (0,)),
            pl.BlockSpec((D,), lambda j: (0,)),
        ],
        out_specs=[
            pl.BlockSpec((512, D), lambda j: (j, 0)),
            pl.BlockSpec((8, 512), lambda j: (0, j)),
        ],
        out_shape=[
            jax.ShapeDtypeStruct((N_PAD, D), jnp.float32),
            jax.ShapeDtypeStruct((8, N_PAD), jnp.float32),
        ],
    )(x_p, W_src, W_dst, att_src, att_dst)


def _mid_body(acch_ref, acce_ref, b_ref, ws_ref, wd_ref, ats_ref, atd_ref,
              h_ref, a2_ref):
    num = acch_ref[...]
    den = acce_ref[:, 0:1]
    y = jnp.maximum(num / (den + 1e-16) + b_ref[...][None, :], 0.0)
    hs = jnp.dot(y, ws_ref[...], preferred_element_type=jnp.float32)
    hd = jnp.dot(y, wd_ref[...], preferred_element_type=jnp.float32)
    h_ref[...] = hs
    a_s = jnp.sum(hs * ats_ref[...][None, :], axis=1)
    a_d = jnp.sum(hd * atd_ref[...][None, :], axis=1)
    a2_ref[...] = jnp.concatenate(
        [a_s[None, :], a_d[None, :], jnp.zeros((6, a_s.shape[0]), jnp.float32)], axis=0)


def _tc_mid(acch, acce, b, W_src, W_dst, att_src, att_dst):
    grid = N_PAD // 512
    return pl.pallas_call(
        _mid_body,
        grid=(grid,),
        in_specs=[
            pl.BlockSpec((512, D), lambda j: (j, 0)),
            pl.BlockSpec((512, DE), lambda j: (j, 0)),
            pl.BlockSpec((D,), lambda j: (0,)),
            pl.BlockSpec((D, D), lambda j: (0, 0)),
            pl.BlockSpec((D, D), lambda j: (0, 0)),
            pl.BlockSpec((D,), lambda j: (0,)),
            pl.BlockSpec((D,), lambda j: (0,)),
        ],
        out_specs=[
            pl.BlockSpec((512, D), lambda j: (j, 0)),
            pl.BlockSpec((8, 512), lambda j: (0, j)),
        ],
        out_shape=[
            jax.ShapeDtypeStruct((N_PAD, D), jnp.float32),
            jax.ShapeDtypeStruct((8, N_PAD), jnp.float32),
        ],
    )(acch, acce, b, W_src, W_dst, att_src, att_dst)


def _final_body(acch_ref, acce_ref, b_ref, out_ref):
    out_ref[...] = (acch_ref[...] / (acce_ref[:, 0:1] + 1e-16)
                    + b_ref[...][None, :])


def _tc_final(acch, acce, b):
    grid = N_PAD // 512
    return pl.pallas_call(
        _final_body,
        grid=(grid,),
        in_specs=[
            pl.BlockSpec((512, D), lambda j: (j, 0)),
            pl.BlockSpec((512, DE), lambda j: (j, 0)),
            pl.BlockSpec((D,), lambda j: (0,)),
        ],
        out_specs=pl.BlockSpec((512, D), lambda j: (j, 0)),
        out_shape=jax.ShapeDtypeStruct((N_PAD, D), jnp.float32),
    )(acch, acce, b)


def _sc_body(h_hbm, a2_hbm, src_hbm, dst_hbm, acch_hbm, acce_hbm,
             asrc_v, adst_v, src_v, dst_v, srcl_v, dstl_v, didx_v, evec_v,
             rows_v, ew_v, acch_sh, acce_sh, sem):
    cid = lax.axis_index("c")
    sid = lax.axis_index("s")
    lo = cid * NHALF

    zero16 = jnp.zeros((16,), jnp.float32)

    def _zero_acc():
        @pl.loop(0, K)
        def _zero(k):
            for u in range(D // 16):
                rows_v[k, pl.ds(u * 16, 16)] = zero16
            ew_v[k, :] = zero16

        for t in range(SUB_PT // K):
            pltpu.sync_copy(rows_v, acch_sh.at[pl.ds(sid * SUB_PT + t * K, K)])
            pltpu.sync_copy(ew_v, acce_sh.at[pl.ds(sid * SUB_PT + t * K, K)])

    _zero_acc()

    pltpu.sync_copy(a2_hbm.at[0], asrc_v)
    pltpu.sync_copy(a2_hbm.at[1], adst_v)

    @pl.loop(0, NBLK, init_carry=(jnp.int32(0), jnp.int32(0)))
    def _block(b, offs):
        pltpu.sync_copy(src_hbm.at[pl.ds(sid * EPT + b * SBLK, SBLK)], src_v)
        pltpu.sync_copy(dst_hbm.at[pl.ds(sid * EPT + b * SBLK, SBLK)], dst_v)

        def _compact(g, offs):
            off_a, off_b = offs
            sv = src_v[pl.ds(g * 16, 16)]
            dv = dst_v[pl.ds(g * 16, 16)]
            dl = dv - lo
            keep_a = (dl >= 0) & (dl < NQ)
            keep_b = (dl >= NQ) & (dl < 2 * NQ)
            cnt_a = plsc.all_reduce_population_count(keep_a)[0]
            cnt_b = plsc.all_reduce_population_count(keep_b)[0]
            plsc.store_compressed(srcl_v.at[pl.ds(off_a, 16)], sv, mask=keep_a)
            plsc.store_compressed(dstl_v.at[pl.ds(off_a, 16)], dl, mask=keep_a)
            off_b = off_b + cnt_b
            start_b = EBUF - off_b
            plsc.store_compressed(srcl_v.at[pl.ds(start_b, 16)], sv, mask=keep_b)
            plsc.store_compressed(dstl_v.at[pl.ds(start_b, 16)], dl - NQ,
                                  mask=keep_b)
            return off_a + cnt_a, off_b

        return pl.loop(0, SBLK // 16, init_carry=offs)(_compact)

    n_a, n_b = _block

    pad_src = jnp.zeros((16,), jnp.int32)
    pad_dst = jnp.full((16,), NQ, jnp.int32)
    for t in range(K // 16):
        srcl_v[pl.ds(n_a + t * 16, 16)] = pad_src
        dstl_v[pl.ds(n_a + t * 16, 16)] = pad_dst
        srcl_v[pl.ds(EBUF - n_b - K + t * 16, 16)] = pad_src
        dstl_v[pl.ds(EBUF - n_b - K + t * 16, 16)] = pad_dst

    plsc.subcore_barrier()

    def _heavy(chunk_base, nch, glob_base):
        @pl.loop(0, nch)
        def _chunk(j):
            base = chunk_base + j * K
            gather = pltpu.async_copy(h_hbm.at[srcl_v.at[pl.ds(base, K)]],
                                      rows_v, sem)
            for i in range(K // 16):
                sv = srcl_v[pl.ds(base + i * 16, 16)]
                dlv = dstl_v[pl.ds(base + i * 16, 16)]
                didx_v[0, pl.ds(i * 16, 16)] = dlv
                dg = jnp.minimum(dlv, NQ - 1) + glob_base
                al = (plsc.load_gather(asrc_v, [sv])
                      + plsc.load_gather(adst_v, [dg]))
                al = jnp.maximum(al, 0.2 * al)
                evec_v[pl.ds(i * 16, 16)] = jnp.exp(al)
            gather.wait()

            @pl.loop(0, K // 16)
            def _scale(g):
                ev = evec_v[pl.ds(g * 16, 16)]
                lane0 = lax.iota(jnp.int32, 16) == 0
                for kk in range(16):
                    e = ev[kk]
                    k = g * 16 + kk
                    for u in range(D // 16):
                        rows_v[k, pl.ds(u * 16, 16)] = rows_v[k, pl.ds(u * 16, 16)] * e
                    ew_v[k, :] = jnp.where(lane0, e, 0.0)

            pltpu.sync_copy(rows_v, acch_sh.at[didx_v.at[0]], add=True)
            pltpu.sync_copy(ew_v, acce_sh.at[didx_v.at[0]], add=True)

    def _copy_out(glob_base):
        pltpu.sync_copy(acch_sh.at[pl.ds(sid * SUB_PT, SUB_PT)],
                        acch_hbm.at[pl.ds(glob_base + sid * SUB_PT, SUB_PT)])
        pltpu.sync_copy(acce_sh.at[pl.ds(sid * SUB_PT, SUB_PT)],
                        acce_hbm.at[pl.ds(glob_base + sid * SUB_PT, SUB_PT)])

    _heavy(jnp.int32(0), (n_a + (K - 1)) // K, lo)
    plsc.subcore_barrier()
    _copy_out(lo)
    _zero_acc()
    plsc.subcore_barrier()

    n_bp = ((n_b + (K - 1)) // K) * K
    _heavy(EBUF - n_bp, n_bp // K, lo + NQ)
    plsc.subcore_barrier()
    _copy_out(lo + NQ)


def _sc_layer(h, a2, src, dst):
    mesh = plsc.VectorSubcoreMesh(core_axis_name="c", subcore_axis_name="s")
    return pl.kernel(
        _sc_body,
        out_type=[
            jax.ShapeDtypeStruct((N_PAD, D), jnp.float32),
            jax.ShapeDtypeStruct((N_PAD, DE), jnp.float32),
        ],
        mesh=mesh,
        compiler_params=pltpu.CompilerParams(needs_layout_passes=False),
        scratch_types=[
            pltpu.VMEM((N_PAD,), jnp.float32),
            pltpu.VMEM((N_PAD,), jnp.float32),
            pltpu.VMEM((SBLK,), jnp.int32),
            pltpu.VMEM((SBLK,), jnp.int32),
            pltpu.VMEM((EBUF,), jnp.int32),
            pltpu.VMEM((EBUF,), jnp.int32),
            pltpu.VMEM((1, K), jnp.int32),
            pltpu.VMEM((K,), jnp.float32),
            pltpu.VMEM((K, D), jnp.float32),
            pltpu.VMEM((K, DE), jnp.float32),
            pltpu.VMEM_SHARED((ROWS_ACC, D), jnp.float32),
            pltpu.VMEM_SHARED((ROWS_ACC, DE), jnp.float32),
            pltpu.SemaphoreType.DMA,
        ],
    )(h, a2, src, dst)


@jax.jit
def _run(x, edge_index, W_src0, W_dst0, att_src0, att_dst0, b0,
         W_src1, W_dst1, att_src1, att_dst1, b1):
    x_p = jnp.pad(x, ((0, N_PAD - N_NODES), (0, 0)))
    src = edge_index[0].astype(jnp.int32)
    dst = edge_index[1].astype(jnp.int32)

    h0, a20 = _tc_prep(x_p, W_src0, W_dst0, att_src0, att_dst0)
    acch0, acce0 = _sc_layer(h0, a20, src, dst)
    h1, a21 = _tc_mid(acch0, acce0, b0, W_src1, W_dst1, att_src1, att_dst1)
    acch1, acce1 = _sc_layer(h1, a21, src, dst)
    out_p = _tc_final(acch1, acce1, b1)
    return out_p[:N_NODES]


def kernel(x, edge_index, W_src0, W_dst0, att_src0, att_dst0, b0,
           W_src1, W_dst1, att_src1, att_dst1, b1):
    return _run(x, edge_index, W_src0, W_dst0, att_src0, att_dst0, b0,
                W_src1, W_dst1, att_src1, att_dst1, b1)
